# baseline (speedup 1.0000x reference)
import functools

import jax
import jax.numpy as jnp
from jax import lax
from jax.experimental import pallas as pl
from jax.experimental.pallas import tpu as pltpu
from jax.experimental.pallas import tpu_sc as plsc

_R, _L = 64, 128
_N = _R * _L
_HALF_ROWS = _R // 2


def _roll(x, s, axis):
    n = x.shape[axis]
    s = s % n
    return jnp.concatenate(
        [lax.slice_in_dim(x, s, n, axis=axis), lax.slice_in_dim(x, 0, s, axis=axis)],
        axis=axis,
    )


def _partner(x, stride, bit_set):
    if stride < _L:
        lo = _roll(x, stride, 2)
        hi = _roll(x, -stride, 2)
    else:
        s = stride // _L
        lo = _roll(x, s, 1)
        hi = _roll(x, -s, 1)
    return jnp.where(bit_set, hi, lo)


def _bitonic_argsort(keys, idx, flat, descending):
    size = 2
    while size <= _N:
        asc = (flat & size) == 0
        stride = size // 2
        while stride >= 1:
            bit_set = (flat & stride) != 0
            kp = _partner(keys, stride, bit_set)
            ip = _partner(idx, stride, bit_set)
            if descending:
                lt = (keys > kp) | ((keys == kp) & (idx < ip))
            else:
                lt = (keys < kp) | ((keys == kp) & (idx < ip))
            want_min = bit_set == jnp.logical_not(asc)
            take_self = lt == want_min
            keys = jnp.where(take_self, keys, kp)
            idx = jnp.where(take_self, idx, ip)
            stride //= 2
        size *= 2
    return idx


def _make_sort_body(descending):
    def body(rand_ref, out_ref):
        v = rand_ref[...]
        batches = v.shape[0]
        bits = lax.bitcast_convert_type(v, jnp.int32)
        m = bits ^ ((bits >> 31) & jnp.int32(0x7FFFFFFF))
        shp = (batches, _R, _L)
        lane = lax.broadcasted_iota(jnp.int32, shp, 2)
        row = lax.broadcasted_iota(jnp.int32, shp, 1)
        bat = lax.broadcasted_iota(jnp.int32, shp, 0)
        flat = row * _L + lane
        gidx = flat + bat * _N
        order = _bitonic_argsort(m, gidx, flat, descending)
        out_ref[...] = order[:, :_HALF_ROWS]

    return body


def _sorted_half(rand3, descending):
    batches = rand3.shape[0]
    out = jax.ShapeDtypeStruct((batches, _HALF_ROWS, _L), jnp.int32)
    return pl.pallas_call(_make_sort_body(descending), out_shape=out)(rand3)


_NC, _NS = 2, 16
_NW = _NC * _NS
_CHUNK = 32
_NB = 2


def _make_gather(out_rows, feat):
    rows_per_w = out_rows // _NW
    n_chunks = rows_per_w // _CHUNK
    mesh = plsc.VectorSubcoreMesh(core_axis_name="c", subcore_axis_name="s")

    @functools.partial(
        pl.kernel,
        mesh=mesh,
        out_type=jax.ShapeDtypeStruct((out_rows, feat), jnp.float32),
        scratch_types=[
            pltpu.VMEM((n_chunks, _CHUNK), jnp.int32),
            [pltpu.VMEM((_CHUNK, feat), jnp.float32)] * _NB,
            [pltpu.SemaphoreType.DMA] * _NB,
            [pltpu.SemaphoreType.DMA] * _NB,
        ],
    )
    def gather(x_hbm, idx_hbm, out_hbm, idx_v, bufs, gsems, wsems):
        wid = lax.axis_index("s") * _NC + lax.axis_index("c")
        base = wid * rows_per_w

        pltpu.sync_copy(idx_hbm.at[pl.ds(wid * n_chunks, n_chunks)], idx_v)

        def start_gather(chunk, b):
            pltpu.async_copy(x_hbm.at[idx_v.at[chunk]], bufs[b], gsems[b])

        def start_write(chunk, b):
            pltpu.async_copy(
                bufs[b], out_hbm.at[pl.ds(base + chunk * _CHUNK, _CHUNK)],
                wsems[b])

        def wait_gather(b):
            pltpu.make_async_copy(x_hbm.at[idx_v.at[0]], bufs[b], gsems[b]).wait()

        def wait_write(b):
            pltpu.make_async_copy(
                bufs[b], out_hbm.at[pl.ds(base, _CHUNK)], wsems[b]).wait()

        for b in range(_NB):
            start_gather(b, b)

        def body(j, carry):
            i0 = j * _NB
            for b in range(_NB):
                wait_gather(b)
                start_write(i0 + b, b)

            @pl.when(j < n_chunks // _NB - 1)
            def _():
                for b in range(_NB):
                    wait_write(b)
                    start_gather(i0 + _NB + b, b)

            return carry

        lax.fori_loop(0, n_chunks // _NB, body, 0)
        for b in range(_NB):
            wait_write(b)

    return gather


def kernel(x, rand_values):
    batches, tokens, feat = x.shape
    rand3 = rand_values.reshape(batches, _R, _L)
    x2d = x.reshape(batches * tokens, feat)
    half_rows = batches * tokens // 2
    gather = _make_gather(half_rows, feat)
    n_chunks_total = half_rows // _CHUNK

    top_g = _sorted_half(rand3, descending=True)
    a2d = gather(x2d, top_g.reshape(n_chunks_total, _CHUNK))
    bot_g = _sorted_half(rand3, descending=False)
    b2d = gather(x2d, bot_g.reshape(n_chunks_total, _CHUNK))

    half = tokens // 2
    return a2d.reshape(batches, half, feat), b2d.reshape(batches, half, feat)


# speedup vs baseline: 1.0312x; 1.0312x over previous
import functools

import jax
import jax.numpy as jnp
from jax import lax
from jax.experimental import pallas as pl
from jax.experimental.pallas import tpu as pltpu
from jax.experimental.pallas import tpu_sc as plsc

_R, _L = 64, 128
_N = _R * _L
_HALF_ROWS = _R // 2


def _roll(x, s, axis):
    n = x.shape[axis]
    s = s % n
    return jnp.concatenate(
        [lax.slice_in_dim(x, s, n, axis=axis), lax.slice_in_dim(x, 0, s, axis=axis)],
        axis=axis,
    )


def _partner(x, stride, bit_set):
    if stride < _L:
        lo = _roll(x, stride, 2)
        hi = _roll(x, -stride, 2)
    else:
        s = stride // _L
        lo = _roll(x, s, 1)
        hi = _roll(x, -s, 1)
    return jnp.where(bit_set, hi, lo)


def _bitonic_argsort(keys, idx, flat, descending):
    size = 2
    while size <= _N:
        asc = (flat & size) == 0
        stride = size // 2
        while stride >= 1:
            bit_set = (flat & stride) != 0
            kp = _partner(keys, stride, bit_set)
            ip = _partner(idx, stride, bit_set)
            if descending:
                lt = (keys > kp) | ((keys == kp) & (idx < ip))
            else:
                lt = (keys < kp) | ((keys == kp) & (idx < ip))
            want_min = bit_set == jnp.logical_not(asc)
            take_self = lt == want_min
            keys = jnp.where(take_self, keys, kp)
            idx = jnp.where(take_self, idx, ip)
            stride //= 2
        size *= 2
    return idx


def _make_sort_body(descending):
    def body(rand_ref, out_ref):
        v = rand_ref[...]
        batches = v.shape[0]
        bits = lax.bitcast_convert_type(v, jnp.int32)
        m = bits ^ ((bits >> 31) & jnp.int32(0x7FFFFFFF))
        shp = (batches, _R, _L)
        lane = lax.broadcasted_iota(jnp.int32, shp, 2)
        row = lax.broadcasted_iota(jnp.int32, shp, 1)
        bat = lax.broadcasted_iota(jnp.int32, shp, 0)
        flat = row * _L + lane
        gidx = flat + bat * _N
        order = _bitonic_argsort(m, gidx, flat, descending)
        out_ref[...] = order[:, :_HALF_ROWS]

    return body


def _sorted_half(rand3, descending):
    batches = rand3.shape[0]
    out = jax.ShapeDtypeStruct((batches, _HALF_ROWS, _L), jnp.int32)
    return pl.pallas_call(_make_sort_body(descending), out_shape=out)(rand3)


_NC, _NS = 2, 16
_NW = _NC * _NS
_CHUNK = 8
_NB = 8


def _make_gather(out_rows, feat):
    rows_per_w = out_rows // _NW
    n_chunks = rows_per_w // _CHUNK
    mesh = plsc.VectorSubcoreMesh(core_axis_name="c", subcore_axis_name="s")

    @functools.partial(
        pl.kernel,
        mesh=mesh,
        out_type=jax.ShapeDtypeStruct((out_rows, feat), jnp.float32),
        scratch_types=[
            pltpu.VMEM((n_chunks, _CHUNK), jnp.int32),
            [pltpu.VMEM((_CHUNK, feat), jnp.float32)] * _NB,
            [pltpu.SemaphoreType.DMA] * _NB,
            [pltpu.SemaphoreType.DMA] * _NB,
        ],
    )
    def gather(x_hbm, idx_hbm, out_hbm, idx_v, bufs, gsems, wsems):
        wid = lax.axis_index("s") * _NC + lax.axis_index("c")
        base = wid * rows_per_w

        pltpu.sync_copy(idx_hbm.at[pl.ds(wid * n_chunks, n_chunks)], idx_v)

        def start_gather(chunk, b):
            pltpu.async_copy(x_hbm.at[idx_v.at[chunk]], bufs[b], gsems[b])

        def start_write(chunk, b):
            pltpu.async_copy(
                bufs[b], out_hbm.at[pl.ds(base + chunk * _CHUNK, _CHUNK)],
                wsems[b])

        def wait_gather(b):
            pltpu.make_async_copy(x_hbm.at[idx_v.at[0]], bufs[b], gsems[b]).wait()

        def wait_write(b):
            pltpu.make_async_copy(
                bufs[b], out_hbm.at[pl.ds(base, _CHUNK)], wsems[b]).wait()

        for b in range(_NB):
            start_gather(b, b)

        def body(j, carry):
            i0 = j * _NB
            for b in range(_NB):
                wait_gather(b)
                start_write(i0 + b, b)

            @pl.when(j < n_chunks // _NB - 1)
            def _():
                for b in range(_NB):
                    wait_write(b)
                    start_gather(i0 + _NB + b, b)

            return carry

        lax.fori_loop(0, n_chunks // _NB, body, 0)
        for b in range(_NB):
            wait_write(b)

    return gather


def kernel(x, rand_values):
    batches, tokens, feat = x.shape
    rand3 = rand_values.reshape(batches, _R, _L)
    x2d = x.reshape(batches * tokens, feat)
    half_rows = batches * tokens // 2
    gather = _make_gather(half_rows, feat)
    n_chunks_total = half_rows // _CHUNK

    top_g = _sorted_half(rand3, descending=True)
    a2d = gather(x2d, top_g.reshape(n_chunks_total, _CHUNK))
    bot_g = _sorted_half(rand3, descending=False)
    b2d = gather(x2d, bot_g.reshape(n_chunks_total, _CHUNK))

    half = tokens // 2
    return a2d.reshape(batches, half, feat), b2d.reshape(batches, half, feat)
